# baseline (speedup 1.0000x reference)
import functools

import jax
import jax.numpy as jnp
from jax import lax
from jax.experimental import pallas as pl
from jax.experimental.pallas import tpu as pltpu
from jax.experimental.pallas import tpu_sc as plsc

NC = 2
NS = 16
L = 16
NW = NC * NS

BATCH = 16384
DIM = 32
B_PER_W = BATCH // NW
IDX_CHUNK = 128
N_CHUNKS = B_PER_W // IDX_CHUNK
GROUPS = B_PER_W // L


def _body(uid_hbm, cid_hbm, umat_hbm, imat_hbm, out_hbm,
          uidx_v, cidx_v, urows_v, irows_v, out_v, sem):
  wid = lax.axis_index("s") * NC + lax.axis_index("c")
  base = wid * B_PER_W

  row0 = wid * N_CHUNKS
  pltpu.sync_copy(uid_hbm.at[pl.ds(row0, N_CHUNKS)], uidx_v)
  pltpu.sync_copy(cid_hbm.at[pl.ds(row0, N_CHUNKS)], cidx_v)

  copies = []
  for j in range(N_CHUNKS):
    copies.append(pltpu.async_copy(
        umat_hbm.at[uidx_v.at[j]], urows_v.at[pl.ds(j * IDX_CHUNK, IDX_CHUNK)],
        sem))
    copies.append(pltpu.async_copy(
        imat_hbm.at[cidx_v.at[j]], irows_v.at[pl.ds(j * IDX_CHUNK, IDX_CHUNK)],
        sem))
  for c in copies:
    c.wait()

  lanes = lax.iota(jnp.int32, L)

  def group(g, _):
    rows = jnp.full((L,), g * L, jnp.int32) + lanes
    acc = jnp.zeros((L,), jnp.float32)
    for d in range(DIM):
      col = jnp.full((L,), d, jnp.int32)
      u = plsc.load_gather(urows_v, [rows, col])
      v = plsc.load_gather(irows_v, [rows, col])
      acc = acc + u * v
    e = jnp.exp(-jnp.abs(acc))
    pos = 1.0 / (1.0 + e)
    neg = e / (1.0 + e)
    sig = jnp.where(acc >= 0, pos, neg)
    out_v[pl.ds(pl.multiple_of(g * L, L), L)] = sig
    return _

  lax.fori_loop(0, GROUPS, group, 0, unroll=False)

  pltpu.sync_copy(out_v, out_hbm.at[pl.ds(base, B_PER_W)])


@jax.jit
def kernel(user_ids, content_ids, user_matrix, item_matrix):
  uid = user_ids.astype(jnp.int32).reshape(BATCH // IDX_CHUNK, IDX_CHUNK)
  cid = content_ids.astype(jnp.int32).reshape(BATCH // IDX_CHUNK, IDX_CHUNK)

  mesh = plsc.VectorSubcoreMesh(
      core_axis_name="c", subcore_axis_name="s", num_cores=NC,
      num_subcores=NS)

  run = pl.kernel(
      _body,
      out_type=jax.ShapeDtypeStruct((BATCH,), jnp.float32),
      mesh=mesh,
      compiler_params=pltpu.CompilerParams(
          needs_layout_passes=False, use_tc_tiling_on_sc=False),
      scratch_types=[
          pltpu.VMEM((N_CHUNKS, IDX_CHUNK), jnp.int32),
          pltpu.VMEM((N_CHUNKS, IDX_CHUNK), jnp.int32),
          pltpu.VMEM((B_PER_W, DIM), jnp.float32),
          pltpu.VMEM((B_PER_W, DIM), jnp.float32),
          pltpu.VMEM((B_PER_W,), jnp.float32),
          pltpu.SemaphoreType.DMA,
      ],
  )
  return run(uid, cid, user_matrix, item_matrix)
